# Initial kernel scaffold; baseline (speedup 1.0000x reference)
import jax
import jax.numpy as jnp
from jax.experimental import pallas as pl


def kernel(x, edge_index, W1, b1, W2, b2):
    raise NotImplementedError("write your pallas kernel here")


# speedup vs baseline: 30.4442x; 30.4442x over previous
import functools

import jax
import jax.numpy as jnp
from jax import lax
from jax.experimental import pallas as pl
from jax.experimental.pallas import tpu as pltpu
from jax.experimental.pallas import tpu_sc as plsc

N = 10000
E = 320000
D = 128
H = 64
C = 16
KSTEPS = 10
ALPHA = 0.1

NW = 32
CHUNK = 128
EW = 10240
NCH = EW // CHUNK
GRP = 8
NGRP = NCH // GRP
EPAD = NW * EW - E
NPAD = 10240
ROWS_PER_TILE = NPAD // 16

_mesh = plsc.VectorSubcoreMesh(core_axis_name="c", subcore_axis_name="s")


@functools.partial(
    pl.kernel,
    mesh=_mesh,
    out_type=[
        jax.ShapeDtypeStruct((NPAD, C), jnp.float32),
        jax.ShapeDtypeStruct((NPAD, C), jnp.float32),
    ],
    scratch_types=[
        pltpu.VMEM((NCH, CHUNK), jnp.int32),
        pltpu.VMEM((NCH, CHUNK), jnp.int32),
        pltpu.VMEM((GRP * CHUNK, C), jnp.float32),
        pltpu.VMEM((ROWS_PER_TILE, C), jnp.float32),
        pltpu.VMEM_SHARED((NPAD, C), jnp.float32),
        pltpu.SemaphoreType.DMA,
    ],
    compiler_params=pltpu.CompilerParams(use_tc_tiling_on_sc=False),
)
def _sc_scatter(u_hbm, src_hbm, dst_hbm, agg0_hbm, agg1_hbm,
                src_t, dst_t, rows_v, zero_v, agg_sp, sem):
    cid = lax.axis_index("c")
    sid = lax.axis_index("s")
    wid = sid * 2 + cid

    pltpu.sync_copy(src_hbm.at[wid], src_t)
    pltpu.sync_copy(dst_hbm.at[wid], dst_t)

    def _zrow(i, _):
        zero_v[i, :] = jnp.zeros((C,), jnp.float32)
        return 0
    lax.fori_loop(0, ROWS_PER_TILE, _zrow, 0)
    tile_rows = pl.ds(sid * ROWS_PER_TILE, ROWS_PER_TILE)
    pltpu.sync_copy(zero_v, agg_sp.at[tile_rows])
    plsc.subcore_barrier()

    def _group(g, _):
        copies = []
        for b in range(GRP):
            cp = pltpu.async_copy(
                u_hbm.at[src_t.at[g * GRP + b]],
                rows_v.at[pl.ds(b * CHUNK, CHUNK)],
                sem,
            )
            copies.append(cp)
        for cp in copies:
            cp.wait()
        for b in range(GRP):
            pltpu.sync_copy(
                rows_v.at[pl.ds(b * CHUNK, CHUNK)],
                agg_sp.at[dst_t.at[g * GRP + b]],
                add=True,
            )
        return 0
    lax.fori_loop(0, NGRP, _group, 0)

    plsc.subcore_barrier()

    @pl.when(cid == 0)
    def _():
        pltpu.sync_copy(agg_sp.at[tile_rows], agg0_hbm.at[tile_rows])

    @pl.when(cid == 1)
    def _():
        pltpu.sync_copy(agg_sp.at[tile_rows], agg1_hbm.at[tile_rows])


ROWB = 400
NBLK = N // ROWB


def _pre_body(x_ref, w1_ref, b1_ref, w2_ref, b2_ref, dga_ref, dgb_ref,
              h_ref, u0_ref, hh_ref, c1_ref, di_ref):
    h1 = jnp.maximum(jnp.dot(x_ref[...], w1_ref[...],
                             preferred_element_type=jnp.float32) + b1_ref[...], 0.0)
    h = jnp.dot(h1, w2_ref[...], preferred_element_type=jnp.float32) + b2_ref[...]
    deg = dga_ref[...] + dgb_ref[...] + 1.0
    dinv = lax.rsqrt(deg)
    h_ref[...] = h
    u0_ref[...] = h * dinv
    hh_ref[...] = (ALPHA) * dinv * h
    c1_ref[...] = (1.0 - ALPHA) * dinv * dinv
    di_ref[...] = dinv


_pre = pl.pallas_call(
    _pre_body,
    grid=(NBLK,),
    in_specs=[
        pl.BlockSpec((ROWB, D), lambda i: (i, 0)),
        pl.BlockSpec((D, H), lambda i: (0, 0)),
        pl.BlockSpec((1, H), lambda i: (0, 0)),
        pl.BlockSpec((H, C), lambda i: (0, 0)),
        pl.BlockSpec((1, C), lambda i: (0, 0)),
        pl.BlockSpec((ROWB, C), lambda i: (i, 0)),
        pl.BlockSpec((ROWB, C), lambda i: (i, 0)),
    ],
    out_specs=[pl.BlockSpec((ROWB, C), lambda i: (i, 0))] * 5,
    out_shape=[jax.ShapeDtypeStruct((N, C), jnp.float32)] * 5,
)


def _combine_body(a0_ref, a1_ref, u_ref, c1_ref, hh_ref, out_ref):
    agg = a0_ref[...] + a1_ref[...] + u_ref[...]
    out_ref[...] = c1_ref[...] * agg + hh_ref[...]


_combine = pl.pallas_call(
    _combine_body,
    grid=(NBLK,),
    in_specs=[
        pl.BlockSpec((ROWB, C), lambda i: (i, 0)),
        pl.BlockSpec((ROWB, C), lambda i: (i, 0)),
        pl.BlockSpec((ROWB, C), lambda i: (i, 0)),
        pl.BlockSpec((ROWB, C), lambda i: (i, 0)),
        pl.BlockSpec((ROWB, C), lambda i: (i, 0)),
    ],
    out_specs=pl.BlockSpec((ROWB, C), lambda i: (i, 0)),
    out_shape=jax.ShapeDtypeStruct((N, C), jnp.float32),
)


def _final_body(a0_ref, a1_ref, u_ref, di_ref, h_ref, out_ref):
    agg = a0_ref[...] + a1_ref[...] + u_ref[...]
    z = (1.0 - ALPHA) * di_ref[...] * agg + ALPHA * h_ref[...]
    m = jnp.max(z, axis=1, keepdims=True)
    ez = jnp.exp(z - m)
    s = jnp.sum(ez, axis=1, keepdims=True)
    out_ref[...] = z - m - jnp.log(s)


_final = pl.pallas_call(
    _final_body,
    grid=(NBLK,),
    in_specs=[
        pl.BlockSpec((ROWB, C), lambda i: (i, 0)),
        pl.BlockSpec((ROWB, C), lambda i: (i, 0)),
        pl.BlockSpec((ROWB, C), lambda i: (i, 0)),
        pl.BlockSpec((ROWB, C), lambda i: (i, 0)),
        pl.BlockSpec((ROWB, C), lambda i: (i, 0)),
    ],
    out_specs=pl.BlockSpec((ROWB, C), lambda i: (i, 0)),
    out_shape=jax.ShapeDtypeStruct((N, C), jnp.float32),
)


def kernel(x, edge_index, W1, b1, W2, b2):
    src = edge_index[0].astype(jnp.int32)
    dst = edge_index[1].astype(jnp.int32)
    padi = jnp.arange(EPAD, dtype=jnp.int32) % CHUNK
    srcp = jnp.concatenate([src, padi]).reshape(NW, NCH, CHUNK)
    dstp = jnp.concatenate([dst, padi + N]).reshape(NW, NCH, CHUNK)

    ones = jnp.ones((N, C), jnp.float32)
    dga, dgb = _sc_scatter(ones, srcp, dstp)

    h, u, hh, c1b, dinvb = _pre(x, W1, b1.reshape(1, H), W2, b2.reshape(1, C),
                                dga[:N], dgb[:N])
    for _ in range(KSTEPS - 1):
        a0, a1 = _sc_scatter(u, srcp, dstp)
        u = _combine(a0[:N], a1[:N], u, c1b, hh)
    a0, a1 = _sc_scatter(u, srcp, dstp)
    return _final(a0[:N], a1[:N], u, dinvb, h)
